# Initial kernel scaffold; baseline (speedup 1.0000x reference)
import jax
import jax.numpy as jnp
from jax.experimental import pallas as pl


def kernel(t, pos, poi_t, poi_pos, batch, W1, b1, W2, b2, W3, b3, W4, b4, W5, b5):
    raise NotImplementedError("write your pallas kernel here")


# speedup vs baseline: 3.7476x; 3.7476x over previous
import functools

import jax
import jax.numpy as jnp
from jax import lax
from jax.experimental import pallas as pl
from jax.experimental.pallas import tpu as pltpu
from jax.experimental.pallas import tpu_sc as plsc

N = 1600000
S = 50000

NC = 2
NS = 16
NW = NC * NS

BLK = 2048
CH = BLK // 128
NB = 25
PPW = BLK * NB
W31_FULL = 6
TAIL = 512
TAIL_CH = TAIL // 128
NT = N // 128

SPAD = 50176
SP_SUB = SPAD // NS
SP_W = SPAD // NW

BRF = 50


def _rsqrt(x):
    xi = plsc.bitcast(x, jnp.int32)
    yi = jnp.int32(0x5F3759DF) - lax.shift_right_logical(xi, 1)
    y = plsc.bitcast(yi, jnp.float32)
    y = y * (1.5 - 0.5 * x * y * y)
    y = y * (1.5 - 0.5 * x * y * y)
    return y


def _splat_i32(v):
    return jnp.full((16,), v, jnp.int32)


def _k1_body(pos_hbm, b2d_hbm, zeros_hbm, part_hbm, posb, rowb, idxb,
             cent_sh):
    c = lax.axis_index("c")
    s = lax.axis_index("s")
    wid = s * NC + c
    iota = lax.iota(jnp.int32, 16)
    pltpu.sync_copy(zeros_hbm, cent_sh.at[pl.ds(s * SP_SUB, SP_SUB)])
    pltpu.sync_copy(zeros_hbm.at[pl.ds(0, BLK)], rowb)
    plsc.subcore_barrier()

    def do_block(k, npts, nch):
        base = wid * PPW + k * BLK
        row0 = wid * (PPW // 128) + k * CH
        pltpu.sync_copy(b2d_hbm.at[pl.ds(row0, nch)], idxb.at[pl.ds(0, nch)])
        pltpu.sync_copy(pos_hbm.at[pl.ds(base, npts)],
                        posb.at[pl.ds(0, npts)])

        def body(i, _):
            rows = i * 16 + iota
            for col in range(3):
                v = plsc.load_gather(posb, [rows, _splat_i32(col)])
                plsc.store_scatter(rowb, [rows, _splat_i32(col)], v)
            return 0

        lax.fori_loop(0, npts // 16, body, 0)
        for j in range(nch):
            pltpu.sync_copy(rowb.at[pl.ds(j * 128, 128)],
                            cent_sh.at[idxb.at[j]], add=True)

    for k in range(NB):
        if k < W31_FULL:
            do_block(k, BLK, CH)
        else:
            @pl.when(wid < NW - 1)
            def _(k=k):
                do_block(k, BLK, CH)

    @pl.when(wid == NW - 1)
    def _():
        do_block(W31_FULL, TAIL, TAIL_CH)

    plsc.subcore_barrier()
    pltpu.sync_copy(cent_sh.at[pl.ds(s * SP_SUB, SP_SUB)],
                    part_hbm.at[c, pl.ds(s * SP_SUB, SP_SUB)])


def _k2_body(part_hbm, poit_hbm, poip_hbm, seg_hbm, pa, pb, pt, pp, ob):
    c = lax.axis_index("c")
    s = lax.axis_index("s")
    wid = s * NC + c
    base = wid * SP_W
    pltpu.sync_copy(part_hbm.at[0, pl.ds(base, SP_W)], pa)
    pltpu.sync_copy(part_hbm.at[1, pl.ds(base, SP_W)], pb)
    pltpu.sync_copy(poit_hbm.at[pl.ds(base, SP_W)], pt)
    pltpu.sync_copy(poip_hbm.at[pl.ds(base, SP_W)], pp)
    iota = lax.iota(jnp.int32, 16)

    def body(i, _):
        r = pl.multiple_of(i * 16, 16)
        rows = i * 16 + iota
        cx = plsc.load_gather(pa, [rows, _splat_i32(0)]) + \
            plsc.load_gather(pb, [rows, _splat_i32(0)])
        cy = plsc.load_gather(pa, [rows, _splat_i32(1)]) + \
            plsc.load_gather(pb, [rows, _splat_i32(1)])
        cz = plsc.load_gather(pa, [rows, _splat_i32(2)]) + \
            plsc.load_gather(pb, [rows, _splat_i32(2)])
        px = plsc.load_gather(pp, [rows, _splat_i32(0)])
        py = plsc.load_gather(pp, [rows, _splat_i32(1)])
        pz = plsc.load_gather(pp, [rows, _splat_i32(2)])
        ptv = pt[pl.ds(r, 16)]
        dcx = cx - px
        dcy = cy - py
        dcz = cz - pz
        dc2 = dcx * dcx + dcy * dcy + dcz * dcz
        for col, val in enumerate((ptv, px, py, pz, dcx, dcy, dcz, dc2)):
            plsc.store_scatter(ob, [rows, _splat_i32(col)], val)
        return 0

    lax.fori_loop(0, SP_W // 16, body, 0)
    pltpu.sync_copy(ob, seg_hbm.at[pl.ds(base, SP_W)])


def _k3_body(t_hbm, pos_hbm, b2d_hbm, seg_hbm, f_hbm, tb, posb, idxb, segb,
             ob):
    c = lax.axis_index("c")
    s = lax.axis_index("s")
    wid = s * NC + c
    iota = lax.iota(jnp.int32, 16)

    def do_block(k, npts, nch):
        base = wid * PPW + k * BLK
        row0 = wid * (PPW // 128) + k * CH
        pltpu.sync_copy(b2d_hbm.at[pl.ds(row0, nch)], idxb.at[pl.ds(0, nch)])
        pltpu.sync_copy(t_hbm.at[pl.ds(base, npts)], tb.at[pl.ds(0, npts)])
        pltpu.sync_copy(pos_hbm.at[pl.ds(base, npts)],
                        posb.at[pl.ds(0, npts)])
        for j in range(nch):
            pltpu.sync_copy(seg_hbm.at[idxb.at[j]],
                            segb.at[pl.ds(j * 128, 128)])

        def body(i, _):
            r = pl.multiple_of(i * 16, 16)
            rows = i * 16 + iota
            jt = i // 8
            l0 = pl.multiple_of((i % 8) * 16, 16)
            tv = tb[pl.ds(r, 16)]
            xv = plsc.load_gather(posb, [rows, _splat_i32(0)])
            yv = plsc.load_gather(posb, [rows, _splat_i32(1)])
            zv = plsc.load_gather(posb, [rows, _splat_i32(2)])
            g_pt = plsc.load_gather(segb, [rows, _splat_i32(0)])
            g_px = plsc.load_gather(segb, [rows, _splat_i32(1)])
            g_py = plsc.load_gather(segb, [rows, _splat_i32(2)])
            g_pz = plsc.load_gather(segb, [rows, _splat_i32(3)])
            g_dx = plsc.load_gather(segb, [rows, _splat_i32(4)])
            g_dy = plsc.load_gather(segb, [rows, _splat_i32(5)])
            g_dz = plsc.load_gather(segb, [rows, _splat_i32(6)])
            g_d2 = plsc.load_gather(segb, [rows, _splat_i32(7)])
            dpx = xv - g_px
            dpy = yv - g_py
            dpz = zv - g_pz
            r2 = dpx * dpx + dpy * dpy + dpz * dpz
            dt = tv - g_pt
            dot = dpx * g_dx + dpy * g_dy + dpz * g_dz
            cos = dot * _rsqrt(jnp.maximum(r2 * g_d2, 1e-30))
            rs = _rsqrt(jnp.maximum(r2, 1e-30))
            ob[jt, 0, pl.ds(l0, 16)] = dt
            ob[jt, 1, pl.ds(l0, 16)] = r2
            ob[jt, 2, pl.ds(l0, 16)] = cos
            ob[jt, 3, pl.ds(l0, 16)] = dpx * rs
            ob[jt, 4, pl.ds(l0, 16)] = dpy * rs
            ob[jt, 5, pl.ds(l0, 16)] = dpz * rs
            return 0

        lax.fori_loop(0, npts // 16, body, 0)
        pltpu.sync_copy(ob.at[pl.ds(0, nch)], f_hbm.at[pl.ds(row0, nch)])

    for k in range(NB):
        if k < W31_FULL:
            do_block(k, BLK, CH)
        else:
            @pl.when(wid < NW - 1)
            def _(k=k):
                do_block(k, BLK, CH)

    @pl.when(wid == NW - 1)
    def _():
        do_block(W31_FULL, TAIL, TAIL_CH)


def _k4_body(f_ref, w1_ref, w2_ref, w3_ref, w4_ref, w5_ref,
             b1_ref, b2_ref, b3_ref, b4_ref, b5_ref, y_ref):
    f = f_ref[...]
    layers = ((w1_ref, b1_ref, 3, 10), (w2_ref, b2_ref, 10, 20),
              (w3_ref, b3_ref, 20, 10), (w4_ref, b4_ref, 10, 5),
              (w5_ref, b5_ref, 5, 1))
    h = [f[:, 0, :], f[:, 1, :], f[:, 2, :]]
    for li, (wref, bref, d_in, d_out) in enumerate(layers):
        nxt = []
        for j in range(d_out):
            acc = h[0] * wref[0, j] + bref[j]
            for k in range(1, d_in):
                acc = acc + h[k] * wref[k, j]
            if li < len(layers) - 1:
                acc = jnp.maximum(acc, 0.0)
            nxt.append(acc)
        h = nxt
    w = h[0]
    for q in range(3):
        y_ref[:, q, :] = w * f[:, 3 + q, :]


def _smem_spec():
    return pl.BlockSpec(memory_space=pltpu.SMEM)


def _k5_body(y_hbm, b2d_hbm, zeros_hbm, part_hbm, yb, rowb, idxb, acc_sh):
    c = lax.axis_index("c")
    s = lax.axis_index("s")
    wid = s * NC + c
    iota = lax.iota(jnp.int32, 16)
    pltpu.sync_copy(zeros_hbm, acc_sh.at[pl.ds(s * SP_SUB, SP_SUB)])
    pltpu.sync_copy(zeros_hbm.at[pl.ds(0, BLK)], rowb)
    plsc.subcore_barrier()

    def do_block(k, npts, nch):
        row0 = wid * (PPW // 128) + k * CH
        pltpu.sync_copy(b2d_hbm.at[pl.ds(row0, nch)], idxb.at[pl.ds(0, nch)])
        pltpu.sync_copy(y_hbm.at[pl.ds(row0, nch), pl.ds(0, 3)],
                        yb.at[pl.ds(0, nch)])

        def body(i, _):
            rows = i * 16 + iota
            jt = i // 8
            l0 = pl.multiple_of((i % 8) * 16, 16)
            for q in range(3):
                vq = yb[jt, q, pl.ds(l0, 16)]
                plsc.store_scatter(rowb, [rows, _splat_i32(q)], vq)
            return 0

        lax.fori_loop(0, npts // 16, body, 0)
        for j in range(nch):
            pltpu.sync_copy(rowb.at[pl.ds(j * 128, 128)],
                            acc_sh.at[idxb.at[j]], add=True)

    for k in range(NB):
        if k < W31_FULL:
            do_block(k, BLK, CH)
        else:
            @pl.when(wid < NW - 1)
            def _(k=k):
                do_block(k, BLK, CH)

    @pl.when(wid == NW - 1)
    def _():
        do_block(W31_FULL, TAIL, TAIL_CH)

    plsc.subcore_barrier()
    pltpu.sync_copy(acc_sh.at[pl.ds(s * SP_SUB, SP_SUB)],
                    part_hbm.at[c, pl.ds(s * SP_SUB, SP_SUB)])


def _k6_body(part_hbm, out_hbm, pa, pb, ob):
    c = lax.axis_index("c")
    s = lax.axis_index("s")
    wid = s * NC + c
    base = wid * SP_W
    pltpu.sync_copy(part_hbm.at[0, pl.ds(base, SP_W)], pa)
    pltpu.sync_copy(part_hbm.at[1, pl.ds(base, SP_W)], pb)
    iota = lax.iota(jnp.int32, 16)

    def body(i, _):
        rows = i * 16 + iota
        for col in range(3):
            v = plsc.load_gather(pa, [rows, _splat_i32(col)]) + \
                plsc.load_gather(pb, [rows, _splat_i32(col)])
            plsc.store_scatter(ob, [rows, _splat_i32(col)], v)
        return 0

    lax.fori_loop(0, SP_W // 16, body, 0)
    pltpu.sync_copy(ob, out_hbm.at[pl.ds(base, SP_W)])


_BUILT = None


def _build_sc_kernels():
    global _BUILT
    if _BUILT is not None:
        return _BUILT
    mesh = plsc.VectorSubcoreMesh(core_axis_name="c", subcore_axis_name="s")
    sck = functools.partial(
        pl.kernel, mesh=mesh,
        compiler_params=pltpu.CompilerParams(use_tc_tiling_on_sc=False,
                                             needs_layout_passes=False))
    k1 = sck(
        _k1_body,
        out_type=jax.ShapeDtypeStruct((NC, SPAD, 8), jnp.float32),
        scratch_types=[
            pltpu.VMEM((BLK, 3), jnp.float32),
            pltpu.VMEM((BLK, 8), jnp.float32),
            pltpu.VMEM((CH, 128), jnp.int32),
            pltpu.VMEM_SHARED((SPAD, 8), jnp.float32),
        ],
    )
    k2 = sck(
        _k2_body,
        out_type=jax.ShapeDtypeStruct((SPAD, 8), jnp.float32),
        scratch_types=[
            pltpu.VMEM((SP_W, 8), jnp.float32),
            pltpu.VMEM((SP_W, 8), jnp.float32),
            pltpu.VMEM((SP_W,), jnp.float32),
            pltpu.VMEM((SP_W, 3), jnp.float32),
            pltpu.VMEM((SP_W, 8), jnp.float32),
        ],
    )
    k3 = sck(
        _k3_body,
        out_type=jax.ShapeDtypeStruct((NT, 8, 128), jnp.float32),
        scratch_types=[
            pltpu.VMEM((BLK,), jnp.float32),
            pltpu.VMEM((BLK, 3), jnp.float32),
            pltpu.VMEM((CH, 128), jnp.int32),
            pltpu.VMEM((BLK, 8), jnp.float32),
            pltpu.VMEM((CH, 8, 128), jnp.float32),
        ],
    )
    k5 = sck(
        _k5_body,
        out_type=jax.ShapeDtypeStruct((NC, SPAD, 8), jnp.float32),
        scratch_types=[
            pltpu.VMEM((CH, 3, 128), jnp.float32),
            pltpu.VMEM((BLK, 8), jnp.float32),
            pltpu.VMEM((CH, 128), jnp.int32),
            pltpu.VMEM_SHARED((SPAD, 8), jnp.float32),
        ],
    )
    k6 = sck(
        _k6_body,
        out_type=jax.ShapeDtypeStruct((SPAD, 3), jnp.float32),
        scratch_types=[
            pltpu.VMEM((SP_W, 8), jnp.float32),
            pltpu.VMEM((SP_W, 8), jnp.float32),
            pltpu.VMEM((SP_W, 3), jnp.float32),
        ],
    )
    _BUILT = (k1, k2, k3, k5, k6)
    return _BUILT


_k4 = pl.pallas_call(
    _k4_body,
    grid=(NT // BRF,),
    in_specs=[
        pl.BlockSpec((BRF, 8, 128), lambda i: (i, 0, 0)),
        _smem_spec(), _smem_spec(), _smem_spec(), _smem_spec(), _smem_spec(),
        _smem_spec(), _smem_spec(), _smem_spec(), _smem_spec(), _smem_spec(),
    ],
    out_specs=pl.BlockSpec((BRF, 8, 128), lambda i: (i, 0, 0)),
    out_shape=jax.ShapeDtypeStruct((NT, 8, 128), jnp.float32),
)


def kernel(t, pos, poi_t, poi_pos, batch, W1, b1, W2, b2, W3, b3, W4, b4,
           W5, b5):
    _k1, _k2, _k3, _k5, _k6 = _build_sc_kernels()
    b2d = batch.astype(jnp.int32).reshape(NT, 128)
    spad = SPAD - S
    poit_p = jnp.concatenate([poi_t, jnp.zeros((spad,), jnp.float32)])
    poip_p = jnp.concatenate([poi_pos, jnp.zeros((spad, 3), jnp.float32)],
                             axis=0)
    zeros8 = jnp.zeros((SP_SUB, 8), jnp.float32)

    part = _k1(pos, b2d, zeros8)
    seg = _k2(part, poit_p, poip_p)
    f = _k3(t, pos, b2d, seg)
    y = _k4(f, W1, W2, W3, W4, W5, b1, b2, b3, b4, b5)
    part2 = _k5(y, b2d, zeros8)
    out = _k6(part2)
    return out[:S]


# speedup vs baseline: 3.9868x; 1.0638x over previous
import functools

import jax
import jax.numpy as jnp
from jax import lax
from jax.experimental import pallas as pl
from jax.experimental.pallas import tpu as pltpu
from jax.experimental.pallas import tpu_sc as plsc

N = 1600000
S = 50000

NC = 2
NS = 16
NW = NC * NS

BLK = 2048
CH = BLK // 128
NB = 25
PPW = BLK * NB
W31_FULL = 6
TAIL = 512
TAIL_CH = TAIL // 128
NT = N // 128

SPAD = 50176
SP_SUB = SPAD // NS
SP_W = SPAD // NW

BRF = 50


def _rsqrt(x):
    xi = plsc.bitcast(x, jnp.int32)
    yi = jnp.int32(0x5F3759DF) - lax.shift_right_logical(xi, 1)
    y = plsc.bitcast(yi, jnp.float32)
    y = y * (1.5 - 0.5 * x * y * y)
    y = y * (1.5 - 0.5 * x * y * y)
    return y


def _splat_i32(v):
    return jnp.full((16,), v, jnp.int32)


def _k1_body(pos_hbm, b2d_hbm, zeros_hbm, part_hbm, posb, rowb, idxb,
             cent_sh):
    c = lax.axis_index("c")
    s = lax.axis_index("s")
    wid = s * NC + c
    iota = lax.iota(jnp.int32, 16)
    pltpu.sync_copy(zeros_hbm, cent_sh.at[pl.ds(s * SP_SUB, SP_SUB)])
    pltpu.sync_copy(zeros_hbm.at[pl.ds(0, BLK)], rowb)
    plsc.subcore_barrier()

    def do_block(k, npts, nch):
        base = wid * PPW + k * BLK
        row0 = wid * (PPW // 128) + k * CH
        pltpu.sync_copy(b2d_hbm.at[pl.ds(row0, nch)], idxb.at[pl.ds(0, nch)])
        pltpu.sync_copy(pos_hbm.at[pl.ds(3 * base, 3 * npts)],
                        posb.at[pl.ds(0, 3 * npts)])

        def body(i, _):
            rows = i * 16 + iota
            w0 = rows * 3
            for col in range(3):
                v = plsc.load_gather(posb, [w0 + col])
                plsc.store_scatter(rowb, [rows, _splat_i32(col)], v)
            return 0

        lax.fori_loop(0, npts // 16, body, 0)
        for j in range(nch):
            pltpu.sync_copy(rowb.at[pl.ds(j * 128, 128)],
                            cent_sh.at[idxb.at[j]], add=True)

    for k in range(NB):
        if k < W31_FULL:
            do_block(k, BLK, CH)
        else:
            @pl.when(wid < NW - 1)
            def _(k=k):
                do_block(k, BLK, CH)

    @pl.when(wid == NW - 1)
    def _():
        do_block(W31_FULL, TAIL, TAIL_CH)

    plsc.subcore_barrier()
    pltpu.sync_copy(cent_sh.at[pl.ds(s * SP_SUB, SP_SUB)],
                    part_hbm.at[c, pl.ds(s * SP_SUB, SP_SUB)])


def _k2_body(part_hbm, poit_hbm, poip_hbm, seg_hbm, pa, pb, pt, pp, ob):
    c = lax.axis_index("c")
    s = lax.axis_index("s")
    wid = s * NC + c
    base = wid * SP_W
    pltpu.sync_copy(part_hbm.at[0, pl.ds(base, SP_W)], pa)
    pltpu.sync_copy(part_hbm.at[1, pl.ds(base, SP_W)], pb)
    pltpu.sync_copy(poit_hbm.at[pl.ds(base, SP_W)], pt)
    pltpu.sync_copy(poip_hbm.at[pl.ds(3 * base, 3 * SP_W)], pp)
    iota = lax.iota(jnp.int32, 16)

    def body(i, _):
        r = pl.multiple_of(i * 16, 16)
        rows = i * 16 + iota
        cx = plsc.load_gather(pa, [rows, _splat_i32(0)]) + \
            plsc.load_gather(pb, [rows, _splat_i32(0)])
        cy = plsc.load_gather(pa, [rows, _splat_i32(1)]) + \
            plsc.load_gather(pb, [rows, _splat_i32(1)])
        cz = plsc.load_gather(pa, [rows, _splat_i32(2)]) + \
            plsc.load_gather(pb, [rows, _splat_i32(2)])
        w0 = rows * 3
        px = plsc.load_gather(pp, [w0])
        py = plsc.load_gather(pp, [w0 + 1])
        pz = plsc.load_gather(pp, [w0 + 2])
        ptv = pt[pl.ds(r, 16)]
        dcx = cx - px
        dcy = cy - py
        dcz = cz - pz
        dc2 = dcx * dcx + dcy * dcy + dcz * dcz
        for col, val in enumerate((ptv, px, py, pz, dcx, dcy, dcz, dc2)):
            plsc.store_scatter(ob, [rows, _splat_i32(col)], val)
        return 0

    lax.fori_loop(0, SP_W // 16, body, 0)
    pltpu.sync_copy(ob, seg_hbm.at[pl.ds(base, SP_W)])


def _k3_body(t_hbm, pos_hbm, b2d_hbm, seg_hbm, f_hbm, tb, posb, idxb, segb,
             ob):
    c = lax.axis_index("c")
    s = lax.axis_index("s")
    wid = s * NC + c
    iota = lax.iota(jnp.int32, 16)

    def do_block(k, npts, nch):
        base = wid * PPW + k * BLK
        row0 = wid * (PPW // 128) + k * CH
        pltpu.sync_copy(b2d_hbm.at[pl.ds(row0, nch)], idxb.at[pl.ds(0, nch)])
        pltpu.sync_copy(t_hbm.at[pl.ds(base, npts)], tb.at[pl.ds(0, npts)])
        pltpu.sync_copy(pos_hbm.at[pl.ds(3 * base, 3 * npts)],
                        posb.at[pl.ds(0, 3 * npts)])
        for j in range(nch):
            pltpu.sync_copy(seg_hbm.at[idxb.at[j]],
                            segb.at[pl.ds(j * 128, 128)])

        def body(i, _):
            r = pl.multiple_of(i * 16, 16)
            rows = i * 16 + iota
            jt = i // 8
            l0 = pl.multiple_of((i % 8) * 16, 16)
            tv = tb[pl.ds(r, 16)]
            w0 = rows * 3
            xv = plsc.load_gather(posb, [w0])
            yv = plsc.load_gather(posb, [w0 + 1])
            zv = plsc.load_gather(posb, [w0 + 2])
            g_pt = plsc.load_gather(segb, [rows, _splat_i32(0)])
            g_px = plsc.load_gather(segb, [rows, _splat_i32(1)])
            g_py = plsc.load_gather(segb, [rows, _splat_i32(2)])
            g_pz = plsc.load_gather(segb, [rows, _splat_i32(3)])
            g_dx = plsc.load_gather(segb, [rows, _splat_i32(4)])
            g_dy = plsc.load_gather(segb, [rows, _splat_i32(5)])
            g_dz = plsc.load_gather(segb, [rows, _splat_i32(6)])
            g_d2 = plsc.load_gather(segb, [rows, _splat_i32(7)])
            dpx = xv - g_px
            dpy = yv - g_py
            dpz = zv - g_pz
            r2 = dpx * dpx + dpy * dpy + dpz * dpz
            dt = tv - g_pt
            dot = dpx * g_dx + dpy * g_dy + dpz * g_dz
            cos = dot * _rsqrt(jnp.maximum(r2 * g_d2, 1e-30))
            rs = _rsqrt(jnp.maximum(r2, 1e-30))
            ob[jt, 0, pl.ds(l0, 16)] = dt
            ob[jt, 1, pl.ds(l0, 16)] = r2
            ob[jt, 2, pl.ds(l0, 16)] = cos
            ob[jt, 3, pl.ds(l0, 16)] = dpx * rs
            ob[jt, 4, pl.ds(l0, 16)] = dpy * rs
            ob[jt, 5, pl.ds(l0, 16)] = dpz * rs
            return 0

        lax.fori_loop(0, npts // 16, body, 0)
        pltpu.sync_copy(ob.at[pl.ds(0, nch)], f_hbm.at[pl.ds(row0, nch)])

    for k in range(NB):
        if k < W31_FULL:
            do_block(k, BLK, CH)
        else:
            @pl.when(wid < NW - 1)
            def _(k=k):
                do_block(k, BLK, CH)

    @pl.when(wid == NW - 1)
    def _():
        do_block(W31_FULL, TAIL, TAIL_CH)


def _k4_body(f_ref, w1_ref, w2_ref, w3_ref, w4_ref, w5_ref,
             b1_ref, b2_ref, b3_ref, b4_ref, b5_ref, y_ref):
    f = f_ref[...]
    layers = ((w1_ref, b1_ref, 3, 10), (w2_ref, b2_ref, 10, 20),
              (w3_ref, b3_ref, 20, 10), (w4_ref, b4_ref, 10, 5),
              (w5_ref, b5_ref, 5, 1))
    h = [f[:, 0, :], f[:, 1, :], f[:, 2, :]]
    for li, (wref, bref, d_in, d_out) in enumerate(layers):
        nxt = []
        for j in range(d_out):
            acc = h[0] * wref[0, j] + bref[j]
            for k in range(1, d_in):
                acc = acc + h[k] * wref[k, j]
            if li < len(layers) - 1:
                acc = jnp.maximum(acc, 0.0)
            nxt.append(acc)
        h = nxt
    w = h[0]
    for q in range(3):
        y_ref[:, q, :] = w * f[:, 3 + q, :]


def _smem_spec():
    return pl.BlockSpec(memory_space=pltpu.SMEM)


def _k5_body(y_hbm, b2d_hbm, zeros_hbm, part_hbm, yb, rowb, idxb, acc_sh):
    c = lax.axis_index("c")
    s = lax.axis_index("s")
    wid = s * NC + c
    iota = lax.iota(jnp.int32, 16)
    pltpu.sync_copy(zeros_hbm, acc_sh.at[pl.ds(s * SP_SUB, SP_SUB)])
    pltpu.sync_copy(zeros_hbm.at[pl.ds(0, BLK)], rowb)
    plsc.subcore_barrier()

    def do_block(k, npts, nch):
        row0 = wid * (PPW // 128) + k * CH
        pltpu.sync_copy(b2d_hbm.at[pl.ds(row0, nch)], idxb.at[pl.ds(0, nch)])
        pltpu.sync_copy(y_hbm.at[pl.ds(row0, nch), pl.ds(0, 3)],
                        yb.at[pl.ds(0, nch)])

        def body(i, _):
            rows = i * 16 + iota
            jt = i // 8
            l0 = pl.multiple_of((i % 8) * 16, 16)
            for q in range(3):
                vq = yb[jt, q, pl.ds(l0, 16)]
                plsc.store_scatter(rowb, [rows, _splat_i32(q)], vq)
            return 0

        lax.fori_loop(0, npts // 16, body, 0)
        for j in range(nch):
            pltpu.sync_copy(rowb.at[pl.ds(j * 128, 128)],
                            acc_sh.at[idxb.at[j]], add=True)

    for k in range(NB):
        if k < W31_FULL:
            do_block(k, BLK, CH)
        else:
            @pl.when(wid < NW - 1)
            def _(k=k):
                do_block(k, BLK, CH)

    @pl.when(wid == NW - 1)
    def _():
        do_block(W31_FULL, TAIL, TAIL_CH)

    plsc.subcore_barrier()
    pltpu.sync_copy(acc_sh.at[pl.ds(s * SP_SUB, SP_SUB)],
                    part_hbm.at[c, pl.ds(s * SP_SUB, SP_SUB)])


def _k6_body(part_hbm, out_hbm, pa, pb, ob):
    c = lax.axis_index("c")
    s = lax.axis_index("s")
    wid = s * NC + c
    base = wid * SP_W
    pltpu.sync_copy(part_hbm.at[0, pl.ds(base, SP_W)], pa)
    pltpu.sync_copy(part_hbm.at[1, pl.ds(base, SP_W)], pb)
    iota = lax.iota(jnp.int32, 16)

    def body(i, _):
        rows = i * 16 + iota
        w0 = rows * 3
        for col in range(3):
            v = plsc.load_gather(pa, [rows, _splat_i32(col)]) + \
                plsc.load_gather(pb, [rows, _splat_i32(col)])
            plsc.store_scatter(ob, [w0 + col], v)
        return 0

    lax.fori_loop(0, SP_W // 16, body, 0)
    pltpu.sync_copy(ob, out_hbm.at[pl.ds(3 * base, 3 * SP_W)])


_BUILT = None


def _build_sc_kernels():
    global _BUILT
    if _BUILT is not None:
        return _BUILT
    mesh = plsc.VectorSubcoreMesh(core_axis_name="c", subcore_axis_name="s")
    sck = functools.partial(
        pl.kernel, mesh=mesh,
        compiler_params=pltpu.CompilerParams(use_tc_tiling_on_sc=False,
                                             needs_layout_passes=False))
    k1 = sck(
        _k1_body,
        out_type=jax.ShapeDtypeStruct((NC, SPAD, 8), jnp.float32),
        scratch_types=[
            pltpu.VMEM((BLK * 3,), jnp.float32),
            pltpu.VMEM((BLK, 8), jnp.float32),
            pltpu.VMEM((CH, 128), jnp.int32),
            pltpu.VMEM_SHARED((SPAD, 8), jnp.float32),
        ],
    )
    k2 = sck(
        _k2_body,
        out_type=jax.ShapeDtypeStruct((SPAD, 8), jnp.float32),
        scratch_types=[
            pltpu.VMEM((SP_W, 8), jnp.float32),
            pltpu.VMEM((SP_W, 8), jnp.float32),
            pltpu.VMEM((SP_W,), jnp.float32),
            pltpu.VMEM((SP_W * 3,), jnp.float32),
            pltpu.VMEM((SP_W, 8), jnp.float32),
        ],
    )
    k3 = sck(
        _k3_body,
        out_type=jax.ShapeDtypeStruct((NT, 8, 128), jnp.float32),
        scratch_types=[
            pltpu.VMEM((BLK,), jnp.float32),
            pltpu.VMEM((BLK * 3,), jnp.float32),
            pltpu.VMEM((CH, 128), jnp.int32),
            pltpu.VMEM((BLK, 8), jnp.float32),
            pltpu.VMEM((CH, 8, 128), jnp.float32),
        ],
    )
    k5 = sck(
        _k5_body,
        out_type=jax.ShapeDtypeStruct((NC, SPAD, 8), jnp.float32),
        scratch_types=[
            pltpu.VMEM((CH, 3, 128), jnp.float32),
            pltpu.VMEM((BLK, 8), jnp.float32),
            pltpu.VMEM((CH, 128), jnp.int32),
            pltpu.VMEM_SHARED((SPAD, 8), jnp.float32),
        ],
    )
    k6 = sck(
        _k6_body,
        out_type=jax.ShapeDtypeStruct((SPAD * 3,), jnp.float32),
        scratch_types=[
            pltpu.VMEM((SP_W, 8), jnp.float32),
            pltpu.VMEM((SP_W, 8), jnp.float32),
            pltpu.VMEM((SP_W * 3,), jnp.float32),
        ],
    )
    _BUILT = (k1, k2, k3, k5, k6)
    return _BUILT


_k4 = pl.pallas_call(
    _k4_body,
    grid=(NT // BRF,),
    in_specs=[
        pl.BlockSpec((BRF, 8, 128), lambda i: (i, 0, 0)),
        _smem_spec(), _smem_spec(), _smem_spec(), _smem_spec(), _smem_spec(),
        _smem_spec(), _smem_spec(), _smem_spec(), _smem_spec(), _smem_spec(),
    ],
    out_specs=pl.BlockSpec((BRF, 8, 128), lambda i: (i, 0, 0)),
    out_shape=jax.ShapeDtypeStruct((NT, 8, 128), jnp.float32),
)


def kernel(t, pos, poi_t, poi_pos, batch, W1, b1, W2, b2, W3, b3, W4, b4,
           W5, b5):
    _k1, _k2, _k3, _k5, _k6 = _build_sc_kernels()
    b2d = batch.astype(jnp.int32).reshape(NT, 128)
    spad = SPAD - S
    posf = pos.reshape(N * 3)
    poit_p = jnp.concatenate([poi_t, jnp.zeros((spad,), jnp.float32)])
    poip_f = jnp.concatenate(
        [poi_pos.reshape(S * 3), jnp.zeros((spad * 3,), jnp.float32)])
    zeros8 = jnp.zeros((SP_SUB, 8), jnp.float32)

    part = _k1(posf, b2d, zeros8)
    seg = _k2(part, poit_p, poip_f)
    f = _k3(t, posf, b2d, seg)
    y = _k4(f, W1, W2, W3, W4, W5, b1, b2, b3, b4, b5)
    part2 = _k5(y, b2d, zeros8)
    out = _k6(part2)
    return out.reshape(SPAD, 3)[:S]


# speedup vs baseline: 11.1781x; 2.8037x over previous
import functools

import jax
import jax.numpy as jnp
from jax import lax
from jax.experimental import pallas as pl
from jax.experimental.pallas import tpu as pltpu
from jax.experimental.pallas import tpu_sc as plsc

N = 1600000
S = 50000

NC = 2
NS = 16
NW = NC * NS

BLK = 2048
CH = BLK // 128
NB = 25
PPW = BLK * NB
W31_FULL = 6
TAIL = 512
TAIL_CH = TAIL // 128
NT = N // 128

SPAD = 50176
SP_SUB = SPAD // NS
SP_W = SPAD // NW

BRF = 50


def _rsqrt(x):
    xi = plsc.bitcast(x, jnp.int32)
    yi = jnp.int32(0x5F3759DF) - lax.shift_right_logical(xi, 1)
    y = plsc.bitcast(yi, jnp.float32)
    y = y * (1.5 - 0.5 * x * y * y)
    y = y * (1.5 - 0.5 * x * y * y)
    return y


def _splat_i32(v):
    return jnp.full((16,), v, jnp.int32)


def _k1_body(xs_hbm, ys_hbm, zs_hbm, b2d_hbm, zeros_hbm, part_hbm, posb,
             rowb, idxb, cent_sh):
    c = lax.axis_index("c")
    s = lax.axis_index("s")
    wid = s * NC + c
    iota = lax.iota(jnp.int32, 16)
    pltpu.sync_copy(zeros_hbm, cent_sh.at[pl.ds(s * SP_SUB, SP_SUB)])
    pltpu.sync_copy(zeros_hbm.at[pl.ds(0, BLK)], rowb)
    plsc.subcore_barrier()

    def do_block(k, npts, nch):
        base = wid * PPW + k * BLK
        row0 = wid * (PPW // 128) + k * CH
        pltpu.sync_copy(b2d_hbm.at[pl.ds(row0, nch)], idxb.at[pl.ds(0, nch)])
        for col, ref in enumerate((xs_hbm, ys_hbm, zs_hbm)):
            pltpu.sync_copy(ref.at[pl.ds(base, npts)],
                            posb.at[col, pl.ds(0, npts)])

        def body(i, _):
            r = pl.multiple_of(i * 16, 16)
            rows = i * 16 + iota
            for col in range(3):
                v = posb[col, pl.ds(r, 16)]
                plsc.store_scatter(rowb, [rows, _splat_i32(col)], v)
            return 0

        lax.fori_loop(0, npts // 16, body, 0)
        for j in range(nch):
            pltpu.sync_copy(rowb.at[pl.ds(j * 128, 128)],
                            cent_sh.at[idxb.at[j]], add=True)

    for k in range(NB):
        if k < W31_FULL:
            do_block(k, BLK, CH)
        else:
            @pl.when(wid < NW - 1)
            def _(k=k):
                do_block(k, BLK, CH)

    @pl.when(wid == NW - 1)
    def _():
        do_block(W31_FULL, TAIL, TAIL_CH)

    plsc.subcore_barrier()
    pltpu.sync_copy(cent_sh.at[pl.ds(s * SP_SUB, SP_SUB)],
                    part_hbm.at[c, pl.ds(s * SP_SUB, SP_SUB)])


def _k2_body(part_hbm, poit_hbm, ppx_hbm, ppy_hbm, ppz_hbm, seg_hbm, pa,
             pb, pt, pp, ob):
    c = lax.axis_index("c")
    s = lax.axis_index("s")
    wid = s * NC + c
    base = wid * SP_W
    pltpu.sync_copy(part_hbm.at[0, pl.ds(base, SP_W)], pa)
    pltpu.sync_copy(part_hbm.at[1, pl.ds(base, SP_W)], pb)
    pltpu.sync_copy(poit_hbm.at[pl.ds(base, SP_W)], pt)
    for col, ref in enumerate((ppx_hbm, ppy_hbm, ppz_hbm)):
        pltpu.sync_copy(ref.at[pl.ds(base, SP_W)], pp.at[col])
    iota = lax.iota(jnp.int32, 16)

    def body(i, _):
        r = pl.multiple_of(i * 16, 16)
        rows = i * 16 + iota
        cx = plsc.load_gather(pa, [rows, _splat_i32(0)]) + \
            plsc.load_gather(pb, [rows, _splat_i32(0)])
        cy = plsc.load_gather(pa, [rows, _splat_i32(1)]) + \
            plsc.load_gather(pb, [rows, _splat_i32(1)])
        cz = plsc.load_gather(pa, [rows, _splat_i32(2)]) + \
            plsc.load_gather(pb, [rows, _splat_i32(2)])
        px = pp[0, pl.ds(r, 16)]
        py = pp[1, pl.ds(r, 16)]
        pz = pp[2, pl.ds(r, 16)]
        ptv = pt[pl.ds(r, 16)]
        dcx = cx - px
        dcy = cy - py
        dcz = cz - pz
        dc2 = dcx * dcx + dcy * dcy + dcz * dcz
        for col, val in enumerate((ptv, px, py, pz, dcx, dcy, dcz, dc2)):
            plsc.store_scatter(ob, [rows, _splat_i32(col)], val)
        return 0

    lax.fori_loop(0, SP_W // 16, body, 0)
    pltpu.sync_copy(ob, seg_hbm.at[pl.ds(base, SP_W)])


def _k3_body(t_hbm, xs_hbm, ys_hbm, zs_hbm, b2d_hbm, seg_hbm, f_hbm, tb,
             posb, idxb, segb, ob):
    c = lax.axis_index("c")
    s = lax.axis_index("s")
    wid = s * NC + c
    iota = lax.iota(jnp.int32, 16)

    def do_block(k, npts, nch):
        base = wid * PPW + k * BLK
        row0 = wid * (PPW // 128) + k * CH
        pltpu.sync_copy(b2d_hbm.at[pl.ds(row0, nch)], idxb.at[pl.ds(0, nch)])
        pltpu.sync_copy(t_hbm.at[pl.ds(base, npts)], tb.at[pl.ds(0, npts)])
        for col, ref in enumerate((xs_hbm, ys_hbm, zs_hbm)):
            pltpu.sync_copy(ref.at[pl.ds(base, npts)],
                            posb.at[col, pl.ds(0, npts)])
        for j in range(nch):
            pltpu.sync_copy(seg_hbm.at[idxb.at[j]],
                            segb.at[pl.ds(j * 128, 128)])

        def body(i, _):
            r = pl.multiple_of(i * 16, 16)
            rows = i * 16 + iota
            jt = i // 8
            l0 = pl.multiple_of((i % 8) * 16, 16)
            tv = tb[pl.ds(r, 16)]
            xv = posb[0, pl.ds(r, 16)]
            yv = posb[1, pl.ds(r, 16)]
            zv = posb[2, pl.ds(r, 16)]
            g_pt = plsc.load_gather(segb, [rows, _splat_i32(0)])
            g_px = plsc.load_gather(segb, [rows, _splat_i32(1)])
            g_py = plsc.load_gather(segb, [rows, _splat_i32(2)])
            g_pz = plsc.load_gather(segb, [rows, _splat_i32(3)])
            g_dx = plsc.load_gather(segb, [rows, _splat_i32(4)])
            g_dy = plsc.load_gather(segb, [rows, _splat_i32(5)])
            g_dz = plsc.load_gather(segb, [rows, _splat_i32(6)])
            g_d2 = plsc.load_gather(segb, [rows, _splat_i32(7)])
            dpx = xv - g_px
            dpy = yv - g_py
            dpz = zv - g_pz
            r2 = dpx * dpx + dpy * dpy + dpz * dpz
            dt = tv - g_pt
            dot = dpx * g_dx + dpy * g_dy + dpz * g_dz
            cos = dot * _rsqrt(jnp.maximum(r2 * g_d2, 1e-30))
            rs = _rsqrt(jnp.maximum(r2, 1e-30))
            ob[jt, 0, pl.ds(l0, 16)] = dt
            ob[jt, 1, pl.ds(l0, 16)] = r2
            ob[jt, 2, pl.ds(l0, 16)] = cos
            ob[jt, 3, pl.ds(l0, 16)] = dpx * rs
            ob[jt, 4, pl.ds(l0, 16)] = dpy * rs
            ob[jt, 5, pl.ds(l0, 16)] = dpz * rs
            return 0

        lax.fori_loop(0, npts // 16, body, 0)
        pltpu.sync_copy(ob.at[pl.ds(0, nch)], f_hbm.at[pl.ds(row0, nch)])

    for k in range(NB):
        if k < W31_FULL:
            do_block(k, BLK, CH)
        else:
            @pl.when(wid < NW - 1)
            def _(k=k):
                do_block(k, BLK, CH)

    @pl.when(wid == NW - 1)
    def _():
        do_block(W31_FULL, TAIL, TAIL_CH)


def _k4_body(f_ref, w1_ref, w2_ref, w3_ref, w4_ref, w5_ref,
             b1_ref, b2_ref, b3_ref, b4_ref, b5_ref, y_ref):
    f = f_ref[...]
    layers = ((w1_ref, b1_ref, 3, 10), (w2_ref, b2_ref, 10, 20),
              (w3_ref, b3_ref, 20, 10), (w4_ref, b4_ref, 10, 5),
              (w5_ref, b5_ref, 5, 1))
    h = [f[:, 0, :], f[:, 1, :], f[:, 2, :]]
    for li, (wref, bref, d_in, d_out) in enumerate(layers):
        nxt = []
        for j in range(d_out):
            acc = h[0] * wref[0, j] + bref[j]
            for k in range(1, d_in):
                acc = acc + h[k] * wref[k, j]
            if li < len(layers) - 1:
                acc = jnp.maximum(acc, 0.0)
            nxt.append(acc)
        h = nxt
    w = h[0]
    for q in range(3):
        y_ref[:, q, :] = w * f[:, 3 + q, :]


def _smem_spec():
    return pl.BlockSpec(memory_space=pltpu.SMEM)


def _k5_body(y_hbm, b2d_hbm, zeros_hbm, part_hbm, yb, rowb, idxb, acc_sh):
    c = lax.axis_index("c")
    s = lax.axis_index("s")
    wid = s * NC + c
    iota = lax.iota(jnp.int32, 16)
    pltpu.sync_copy(zeros_hbm, acc_sh.at[pl.ds(s * SP_SUB, SP_SUB)])
    pltpu.sync_copy(zeros_hbm.at[pl.ds(0, BLK)], rowb)
    plsc.subcore_barrier()

    def do_block(k, npts, nch):
        row0 = wid * (PPW // 128) + k * CH
        pltpu.sync_copy(b2d_hbm.at[pl.ds(row0, nch)], idxb.at[pl.ds(0, nch)])
        pltpu.sync_copy(y_hbm.at[pl.ds(row0, nch), pl.ds(0, 3)],
                        yb.at[pl.ds(0, nch)])

        def body(i, _):
            rows = i * 16 + iota
            jt = i // 8
            l0 = pl.multiple_of((i % 8) * 16, 16)
            for q in range(3):
                vq = yb[jt, q, pl.ds(l0, 16)]
                plsc.store_scatter(rowb, [rows, _splat_i32(q)], vq)
            return 0

        lax.fori_loop(0, npts // 16, body, 0)
        for j in range(nch):
            pltpu.sync_copy(rowb.at[pl.ds(j * 128, 128)],
                            acc_sh.at[idxb.at[j]], add=True)

    for k in range(NB):
        if k < W31_FULL:
            do_block(k, BLK, CH)
        else:
            @pl.when(wid < NW - 1)
            def _(k=k):
                do_block(k, BLK, CH)

    @pl.when(wid == NW - 1)
    def _():
        do_block(W31_FULL, TAIL, TAIL_CH)

    plsc.subcore_barrier()
    pltpu.sync_copy(acc_sh.at[pl.ds(s * SP_SUB, SP_SUB)],
                    part_hbm.at[c, pl.ds(s * SP_SUB, SP_SUB)])


def _k6_body(part_hbm, out_hbm, pa, pb, ob):
    c = lax.axis_index("c")
    s = lax.axis_index("s")
    wid = s * NC + c
    base = wid * SP_W
    pltpu.sync_copy(part_hbm.at[0, pl.ds(base, SP_W)], pa)
    pltpu.sync_copy(part_hbm.at[1, pl.ds(base, SP_W)], pb)
    iota = lax.iota(jnp.int32, 16)

    def body(i, _):
        rows = i * 16 + iota
        w0 = rows * 3
        for col in range(3):
            v = plsc.load_gather(pa, [rows, _splat_i32(col)]) + \
                plsc.load_gather(pb, [rows, _splat_i32(col)])
            plsc.store_scatter(ob, [w0 + col], v)
        return 0

    lax.fori_loop(0, SP_W // 16, body, 0)
    pltpu.sync_copy(ob, out_hbm.at[pl.ds(3 * base, 3 * SP_W)])


_BUILT = None


def _build_sc_kernels():
    global _BUILT
    if _BUILT is not None:
        return _BUILT
    mesh = plsc.VectorSubcoreMesh(core_axis_name="c", subcore_axis_name="s")
    sck = functools.partial(
        pl.kernel, mesh=mesh,
        compiler_params=pltpu.CompilerParams(use_tc_tiling_on_sc=False,
                                             needs_layout_passes=False))
    k1 = sck(
        _k1_body,
        out_type=jax.ShapeDtypeStruct((NC, SPAD, 8), jnp.float32),
        scratch_types=[
            pltpu.VMEM((3, BLK), jnp.float32),
            pltpu.VMEM((BLK, 8), jnp.float32),
            pltpu.VMEM((CH, 128), jnp.int32),
            pltpu.VMEM_SHARED((SPAD, 8), jnp.float32),
        ],
    )
    k2 = sck(
        _k2_body,
        out_type=jax.ShapeDtypeStruct((SPAD, 8), jnp.float32),
        scratch_types=[
            pltpu.VMEM((SP_W, 8), jnp.float32),
            pltpu.VMEM((SP_W, 8), jnp.float32),
            pltpu.VMEM((SP_W,), jnp.float32),
            pltpu.VMEM((3, SP_W), jnp.float32),
            pltpu.VMEM((SP_W, 8), jnp.float32),
        ],
    )
    k3 = sck(
        _k3_body,
        out_type=jax.ShapeDtypeStruct((NT, 8, 128), jnp.float32),
        scratch_types=[
            pltpu.VMEM((BLK,), jnp.float32),
            pltpu.VMEM((3, BLK), jnp.float32),
            pltpu.VMEM((CH, 128), jnp.int32),
            pltpu.VMEM((BLK, 8), jnp.float32),
            pltpu.VMEM((CH, 8, 128), jnp.float32),
        ],
    )
    k5 = sck(
        _k5_body,
        out_type=jax.ShapeDtypeStruct((NC, SPAD, 8), jnp.float32),
        scratch_types=[
            pltpu.VMEM((CH, 3, 128), jnp.float32),
            pltpu.VMEM((BLK, 8), jnp.float32),
            pltpu.VMEM((CH, 128), jnp.int32),
            pltpu.VMEM_SHARED((SPAD, 8), jnp.float32),
        ],
    )
    k6 = sck(
        _k6_body,
        out_type=jax.ShapeDtypeStruct((SPAD * 3,), jnp.float32),
        scratch_types=[
            pltpu.VMEM((SP_W, 8), jnp.float32),
            pltpu.VMEM((SP_W, 8), jnp.float32),
            pltpu.VMEM((SP_W * 3,), jnp.float32),
        ],
    )
    _BUILT = (k1, k2, k3, k5, k6)
    return _BUILT


_k4 = pl.pallas_call(
    _k4_body,
    grid=(NT // BRF,),
    in_specs=[
        pl.BlockSpec((BRF, 8, 128), lambda i: (i, 0, 0)),
        _smem_spec(), _smem_spec(), _smem_spec(), _smem_spec(), _smem_spec(),
        _smem_spec(), _smem_spec(), _smem_spec(), _smem_spec(), _smem_spec(),
    ],
    out_specs=pl.BlockSpec((BRF, 8, 128), lambda i: (i, 0, 0)),
    out_shape=jax.ShapeDtypeStruct((NT, 8, 128), jnp.float32),
)


def kernel(t, pos, poi_t, poi_pos, batch, W1, b1, W2, b2, W3, b3, W4, b4,
           W5, b5):
    _k1, _k2, _k3, _k5, _k6 = _build_sc_kernels()
    b2d = batch.astype(jnp.int32).reshape(NT, 128)
    spad = SPAD - S
    xs, ys, zs = pos[:, 0], pos[:, 1], pos[:, 2]
    zs_pad = jnp.zeros((spad,), jnp.float32)
    poit_p = jnp.concatenate([poi_t, zs_pad])
    ppx = jnp.concatenate([poi_pos[:, 0], zs_pad])
    ppy = jnp.concatenate([poi_pos[:, 1], zs_pad])
    ppz = jnp.concatenate([poi_pos[:, 2], zs_pad])
    zeros8 = jnp.zeros((SP_SUB, 8), jnp.float32)

    part = _k1(xs, ys, zs, b2d, zeros8)
    seg = _k2(part, poit_p, ppx, ppy, ppz)
    f = _k3(t, xs, ys, zs, b2d, seg)
    y = _k4(f, W1, W2, W3, W4, W5, b1, b2, b3, b4, b5)
    part2 = _k5(y, b2d, zeros8)
    out = _k6(part2)
    return out.reshape(SPAD, 3)[:S]
